# baseline (speedup 1.0000x reference)
import functools

import jax
import jax.numpy as jnp
from jax import lax
from jax.experimental import pallas as pl
from jax.experimental.pallas import tpu as pltpu
from jax.experimental.pallas import tpu_sc as plsc

B, N, D, R = 128, 1024, 384, 512
TOTAL = B * N
V = R * R

_info = plsc.get_sparse_core_info()
NC, NS, L = _info.num_cores, _info.num_subcores, _info.num_lanes
NW = NC * NS
W = TOTAL // NW
C = 128
NCHUNK = W // C


def _sc_gather_add(x2, xi, yi, table2):
    mesh = plsc.VectorSubcoreMesh(core_axis_name="c", subcore_axis_name="s")

    @functools.partial(
        pl.kernel,
        mesh=mesh,
        out_type=jax.ShapeDtypeStruct((TOTAL, D), jnp.float32),
        scratch_types=[
            pltpu.VMEM((W,), jnp.int32),
            pltpu.VMEM((W,), jnp.int32),
            pltpu.VMEM((W,), jnp.int32),
            pltpu.VMEM((C, D), jnp.float32),
            pltpu.VMEM((C, D), jnp.float32),
            pltpu.SemaphoreType.DMA,
        ],
    )
    def k(x_hbm, xi_hbm, yi_hbm, tab_hbm, out_hbm, xi_v, yi_v, idx_v, xbuf, rbuf, sem):
        wid = lax.axis_index("s") * NC + lax.axis_index("c")
        base = wid * W
        pltpu.sync_copy(xi_hbm.at[pl.ds(base, W)], xi_v)
        pltpu.sync_copy(yi_hbm.at[pl.ds(base, W)], yi_v)

        def compute_idx(t, carry):
            s = pl.ds(t * L, L)
            idx_v[s] = yi_v[s] * R + xi_v[s]
            return carry

        lax.fori_loop(0, W // L, compute_idx, 0)

        def do_chunk(g, carry):
            row0 = base + g * C
            pltpu.sync_copy(x_hbm.at[pl.ds(row0, C)], xbuf)
            pltpu.async_copy(tab_hbm.at[idx_v.at[pl.ds(g * C, C)]], rbuf, sem).wait()

            def add_row(r, c2):
                for j in range(D // L):
                    s = pl.ds(j * L, L)
                    xbuf[r, s] = xbuf[r, s] + rbuf[r, s]
                return c2

            lax.fori_loop(0, C, add_row, 0)
            pltpu.sync_copy(xbuf, out_hbm.at[pl.ds(row0, C)])
            return carry

        lax.fori_loop(0, NCHUNK, do_chunk, 0)

    return k(x2, xi, yi, table2)


def kernel(x, offgrid_coords, pos_table):
    x2 = x.reshape(TOTAL, D)
    xi = offgrid_coords[..., 0].reshape(TOTAL)
    yi = offgrid_coords[..., 1].reshape(TOTAL)
    table2 = pos_table.reshape(V, D)
    out = _sc_gather_add(x2, xi, yi, table2)
    return out.reshape(B, N, D)


# speedup vs baseline: 1.5910x; 1.5910x over previous
import functools

import jax
import jax.numpy as jnp
from jax import lax
from jax.experimental import pallas as pl
from jax.experimental.pallas import tpu as pltpu
from jax.experimental.pallas import tpu_sc as plsc

B, N, D, R = 128, 1024, 384, 512
TOTAL = B * N
V = R * R

_info = plsc.get_sparse_core_info()
NC, NS, L = _info.num_cores, _info.num_subcores, _info.num_lanes
NW = NC * NS
W = TOTAL // NW
C = 64
NCHUNK = W // C


def _sc_gather_add(x2, xi, yi, table2):
    mesh = plsc.VectorSubcoreMesh(core_axis_name="c", subcore_axis_name="s")

    @functools.partial(
        pl.kernel,
        mesh=mesh,
        out_type=jax.ShapeDtypeStruct((TOTAL, D), jnp.float32),
        scratch_types=[
            pltpu.VMEM((W,), jnp.int32),
            pltpu.VMEM((W,), jnp.int32),
            pltpu.VMEM((W,), jnp.int32),
            pltpu.VMEM((C, D), jnp.float32),
            pltpu.VMEM((C, D), jnp.float32),
            pltpu.VMEM((C, D), jnp.float32),
            pltpu.VMEM((C, D), jnp.float32),
            pltpu.SemaphoreType.DMA,
            pltpu.SemaphoreType.DMA,
            pltpu.SemaphoreType.DMA,
            pltpu.SemaphoreType.DMA,
            pltpu.SemaphoreType.DMA,
            pltpu.SemaphoreType.DMA,
        ],
    )
    def k(x_hbm, xi_hbm, yi_hbm, tab_hbm, out_hbm, xi_v, yi_v, idx_v,
          xb0, xb1, rb0, rb1, is0, is1, gs0, gs1, os0, os1):
        xbufs = (xb0, xb1)
        rbufs = (rb0, rb1)
        in_s = (is0, is1)
        ga_s = (gs0, gs1)
        out_s = (os0, os1)
        wid = lax.axis_index("s") * NC + lax.axis_index("c")
        base = wid * W
        pltpu.sync_copy(xi_hbm.at[pl.ds(base, W)], xi_v)
        pltpu.sync_copy(yi_hbm.at[pl.ds(base, W)], yi_v)

        def compute_idx(t, carry):
            s = pl.ds(t * L, L)
            idx_v[s] = yi_v[s] * R + xi_v[s]
            return carry

        lax.fori_loop(0, W // L, compute_idx, 0)

        def start_in(c, b):
            return pltpu.async_copy(x_hbm.at[pl.ds(base + c * C, C)], xbufs[b], in_s[b])

        def start_ga(c, b):
            return pltpu.async_copy(
                tab_hbm.at[idx_v.at[pl.ds(c * C, C)]], rbufs[b], ga_s[b])

        def wait_in(c, b):
            pltpu.make_async_copy(
                x_hbm.at[pl.ds(base + c * C, C)], xbufs[b], in_s[b]).wait()

        def wait_ga(c, b):
            pltpu.make_async_copy(
                tab_hbm.at[idx_v.at[pl.ds(c * C, C)]], rbufs[b], ga_s[b]).wait()

        def add_chunk(b):
            xb, rb = xbufs[b], rbufs[b]

            def add_row(r, carry):
                for j in range(D // L):
                    s = pl.ds(j * L, L)
                    plsc.addupdate(xb.at[r, s], rb[r, s])
                return carry

            lax.fori_loop(0, C, add_row, 0)

        def start_out(c, b):
            return pltpu.async_copy(
                xbufs[b], out_hbm.at[pl.ds(base + c * C, C)], out_s[b])

        start_in(0, 0)
        start_ga(0, 0)
        start_in(1, 1)
        start_ga(1, 1)

        def pair_body(p, carry):
            for b in range(2):
                c = 2 * p + b
                wait_in(c, b)
                wait_ga(c, b)
                add_chunk(b)
                start_out(c, b).wait()
                start_in(c + 2, b)
                start_ga(c + 2, b)
            return carry

        lax.fori_loop(0, NCHUNK // 2 - 1, pair_body, 0)

        for c in (NCHUNK - 2, NCHUNK - 1):
            b = c % 2
            wait_in(c, b)
            wait_ga(c, b)
            add_chunk(b)
            start_out(c, b).wait()

    return k(x2, xi, yi, table2)


def kernel(x, offgrid_coords, pos_table):
    x2 = x.reshape(TOTAL, D)
    xi = offgrid_coords[..., 0].reshape(TOTAL)
    yi = offgrid_coords[..., 1].reshape(TOTAL)
    table2 = pos_table.reshape(V, D)
    out = _sc_gather_add(x2, xi, yi, table2)
    return out.reshape(B, N, D)


# speedup vs baseline: 1.6189x; 1.0175x over previous
import functools

import jax
import jax.numpy as jnp
from jax import lax
from jax.experimental import pallas as pl
from jax.experimental.pallas import tpu as pltpu
from jax.experimental.pallas import tpu_sc as plsc

B, N, D, R = 128, 1024, 384, 512
TOTAL = B * N
V = R * R

_info = plsc.get_sparse_core_info()
NC, NS, L = _info.num_cores, _info.num_subcores, _info.num_lanes
NW = NC * NS
W = TOTAL // NW
C = 32
NCHUNK = W // C
NBUF = 4
K = NBUF - 1


def _sc_gather_add(x2, xi, yi, table2):
    mesh = plsc.VectorSubcoreMesh(core_axis_name="c", subcore_axis_name="s")

    @functools.partial(
        pl.kernel,
        mesh=mesh,
        out_type=jax.ShapeDtypeStruct((TOTAL, D), jnp.float32),
        scratch_types=(
            [pltpu.VMEM((W,), jnp.int32)] * 3
            + [pltpu.VMEM((C, D), jnp.float32)] * NBUF
            + [pltpu.VMEM((C, D), jnp.float32)] * NBUF
            + [pltpu.SemaphoreType.DMA] * (3 * NBUF)
        ),
    )
    def k(x_hbm, xi_hbm, yi_hbm, tab_hbm, out_hbm, xi_v, yi_v, idx_v, *bufs):
        xbufs = bufs[0:NBUF]
        rbufs = bufs[NBUF:2 * NBUF]
        in_s = bufs[2 * NBUF:3 * NBUF]
        ga_s = bufs[3 * NBUF:4 * NBUF]
        out_s = bufs[4 * NBUF:5 * NBUF]
        wid = lax.axis_index("s") * NC + lax.axis_index("c")
        base = wid * W
        pltpu.sync_copy(xi_hbm.at[pl.ds(base, W)], xi_v)
        pltpu.sync_copy(yi_hbm.at[pl.ds(base, W)], yi_v)

        def compute_idx(t, carry):
            s = pl.ds(t * L, L)
            idx_v[s] = yi_v[s] * R + xi_v[s]
            return carry

        lax.fori_loop(0, W // L, compute_idx, 0)

        def start_in(c, b):
            return pltpu.async_copy(x_hbm.at[pl.ds(base + c * C, C)], xbufs[b], in_s[b])

        def start_ga(c, b):
            return pltpu.async_copy(
                tab_hbm.at[idx_v.at[pl.ds(c * C, C)]], rbufs[b], ga_s[b])

        def wait_in(c, b):
            pltpu.make_async_copy(
                x_hbm.at[pl.ds(base + c * C, C)], xbufs[b], in_s[b]).wait()

        def wait_ga(c, b):
            pltpu.make_async_copy(
                tab_hbm.at[idx_v.at[pl.ds(c * C, C)]], rbufs[b], ga_s[b]).wait()

        def start_out(c, b):
            return pltpu.async_copy(
                xbufs[b], out_hbm.at[pl.ds(base + c * C, C)], out_s[b])

        def wait_out(c, b):
            pltpu.make_async_copy(
                xbufs[b], out_hbm.at[pl.ds(base + c * C, C)], out_s[b]).wait()

        def add_chunk(b):
            xb, rb = xbufs[b], rbufs[b]

            def add_row(r, carry):
                for j in range(D // L):
                    s = pl.ds(j * L, L)
                    plsc.addupdate(xb.at[r, s], rb[r, s])
                return carry

            lax.fori_loop(0, C, add_row, 0)

        for c0 in range(K):
            start_in(c0, c0)
            start_ga(c0, c0)

        wait_in(0, 0)
        wait_ga(0, 0)
        add_chunk(0)
        start_out(0, 0)
        start_in(K, K % NBUF)
        start_ga(K, K % NBUF)

        def quad_body(q, carry):
            for j in range(NBUF):
                c = NBUF * q + 1 + j
                b = (1 + j) % NBUF
                wait_in(c, b)
                wait_ga(c, b)
                add_chunk(b)
                start_out(c, b)
                bp = (b + K) % NBUF
                wait_out(c - 1, bp)
                start_in(c + K, bp)
                start_ga(c + K, bp)
            return carry

        lax.fori_loop(0, (NCHUNK - NBUF) // NBUF, quad_body, 0)

        for c in range(NCHUNK - K, NCHUNK):
            b = c % NBUF
            wait_in(c, b)
            wait_ga(c, b)
            add_chunk(b)
            start_out(c, b)
        for c in range(NCHUNK - NBUF, NCHUNK):
            wait_out(c, c % NBUF)

    return k(x2, xi, yi, table2)


def kernel(x, offgrid_coords, pos_table):
    x2 = x.reshape(TOTAL, D)
    xi = offgrid_coords[..., 0].reshape(TOTAL)
    yi = offgrid_coords[..., 1].reshape(TOTAL)
    table2 = pos_table.reshape(V, D)
    out = _sc_gather_add(x2, xi, yi, table2)
    return out.reshape(B, N, D)


# speedup vs baseline: 1.8124x; 1.1195x over previous
import functools

import jax
import jax.numpy as jnp
from jax import lax
from jax.experimental import pallas as pl
from jax.experimental.pallas import tpu as pltpu

B, N, D, R = 128, 1024, 384, 512
TOTAL = B * N
H = D // 2
TB = 1024
GRID = TOTAL // TB


def _tc_body(idx_ref, t1_ref, x_ref, out_ref):
    t1 = t1_ref[...]
    yv = idx_ref[0, 0]
    xv = idx_ref[1, 0]
    iota = lax.broadcasted_iota(jnp.int32, (R, TB), 0)
    ohy = (iota == yv).astype(jnp.bfloat16)
    ohx = (iota == xv).astype(jnp.bfloat16)
    posh = lax.dot_general(
        ohy, t1, dimension_numbers=(((0,), (0,)), ((), ())),
        preferred_element_type=jnp.float32)
    posw = lax.dot_general(
        ohx, t1, dimension_numbers=(((0,), (0,)), ((), ())),
        preferred_element_type=jnp.float32)
    pos = jnp.concatenate([posh, posw], axis=-1)
    out_ref[...] = x_ref[...] + pos


def _tc_add(x2, idx_yx, t1_bf):
    return pl.pallas_call(
        _tc_body,
        grid=(GRID,),
        in_specs=[
            pl.BlockSpec((2, 1, 1, TB), lambda i: (0, i, 0, 0)),
            pl.BlockSpec((R, H), lambda i: (0, 0)),
            pl.BlockSpec((TB, D), lambda i: (i, 0)),
        ],
        out_specs=pl.BlockSpec((TB, D), lambda i: (i, 0)),
        out_shape=jax.ShapeDtypeStruct((TOTAL, D), jnp.float32),
    )(idx_yx, t1_bf, x2)


def kernel(x, offgrid_coords, pos_table):
    x2 = x.reshape(TOTAL, D)
    xi = offgrid_coords[..., 0].reshape(GRID, 1, TB)
    yi = offgrid_coords[..., 1].reshape(GRID, 1, TB)
    idx_yx = jnp.stack([yi, xi], axis=0)
    t1_bf = pos_table[0, :, H:].astype(jnp.bfloat16)
    out = _tc_add(x2, idx_yx, t1_bf)
    return out.reshape(B, N, D)
